# Initial kernel scaffold; baseline (speedup 1.0000x reference)
import jax
import jax.numpy as jnp
from jax.experimental import pallas as pl


def kernel(values, indices):
    raise NotImplementedError("write your pallas kernel here")


# speedup vs baseline: 111.6270x; 111.6270x over previous
import functools
import math

import jax
import jax.numpy as jnp
import numpy as np
from jax import lax
from jax.experimental import pallas as pl
from jax.experimental.pallas import tpu as pltpu
from jax.experimental.pallas import tpu_sc as plsc

_NNZ = 2684354
_PROB = 0.5
_NW = 32
_LANES = 16
_NSUB = 4


def _build_constants():
    u = np.asarray(
        jax.random.uniform(jax.random.key(42), (_NNZ,), dtype=jnp.float32))
    mask = np.floor(u + (1.0 - _PROB)).astype(bool)
    keep = np.flatnonzero(mask).astype(np.int32)
    k = int(keep.size)
    chunk = math.ceil(k / _NW / 64) * 64
    kpad = chunk * _NW
    pad = np.full(kpad - k, keep[-1], np.int32)
    keep0 = np.concatenate([keep, pad])
    keepcat = np.concatenate([keep0, keep0 + _NNZ])
    return keepcat, k, kpad, chunk


_KEEPCAT, _K, _KPAD, _CHUNK = _build_constants()
_SUB = _CHUNK // _NSUB
assert _SUB % _LANES == 0


@jax.jit
def _sc_dropout(values, ind_flat, keepcat):
    mesh = plsc.VectorSubcoreMesh(core_axis_name="c", subcore_axis_name="s")

    @functools.partial(
        pl.kernel,
        out_type=[
            jax.ShapeDtypeStruct((2 * _KPAD,), jnp.int32),
            jax.ShapeDtypeStruct((_KPAD,), jnp.float32),
        ],
        mesh=mesh,
        scratch_types=[
            pltpu.VMEM((_SUB,), jnp.int32),
            pltpu.VMEM((_SUB,), jnp.float32),
            pltpu.VMEM((_SUB,), jnp.int32),
            pltpu.SemaphoreType.DMA,
        ],
    )
    def k(values_hbm, ind_hbm, keep_hbm, outi_hbm, outv_hbm,
          idx_v, vbuf, ibuf, sem):
        wid = lax.axis_index("s") * 2 + lax.axis_index("c")

        def sub(j, carry):
            base = wid * _CHUNK + j * _SUB
            pltpu.sync_copy(keep_hbm.at[pl.ds(base, _SUB)], idx_v)
            pltpu.async_copy(values_hbm.at[idx_v], vbuf, sem).wait()

            def scale(i, c):
                sl = pl.ds(i * _LANES, _LANES)
                vbuf[sl] = vbuf[sl] * 2.0
                return c

            lax.fori_loop(0, _SUB // _LANES, scale, 0, unroll=4)
            pltpu.sync_copy(vbuf, outv_hbm.at[pl.ds(base, _SUB)])
            pltpu.async_copy(ind_hbm.at[idx_v], ibuf, sem).wait()
            pltpu.sync_copy(ibuf, outi_hbm.at[pl.ds(base, _SUB)])
            pltpu.sync_copy(keep_hbm.at[pl.ds(_KPAD + base, _SUB)], idx_v)
            pltpu.async_copy(ind_hbm.at[idx_v], ibuf, sem).wait()
            pltpu.sync_copy(ibuf, outi_hbm.at[pl.ds(_KPAD + base, _SUB)])
            return carry

        lax.fori_loop(0, _NSUB, sub, 0)

    return k(values, ind_flat, keepcat)


def kernel(values, indices):
    outi_flat, outv = _sc_dropout(values, indices.reshape(-1),
                                  jnp.asarray(_KEEPCAT))
    idx = outi_flat.reshape(2, _KPAD)[:, :_K]
    val = outv[:_K]
    return idx, val


# speedup vs baseline: 656.9619x; 5.8853x over previous
import functools
import math

import jax
import jax.numpy as jnp
import numpy as np
from jax import lax
from jax.experimental import pallas as pl
from jax.experimental.pallas import tpu as pltpu
from jax.experimental.pallas import tpu_sc as plsc

_NNZ = 2684354
_PROB = 0.5
_NW = 32
_LANES = 16
_NSUB = 4


def _span_start(b, d, s_max):
    s = 2 * b - d
    s = jnp.minimum(jnp.maximum(s, 0), s_max) if not isinstance(b, int) \
        else min(max(s, 0), s_max)
    return s


def _build_constants():
    u = np.asarray(
        jax.random.uniform(jax.random.key(42), (_NNZ,), dtype=jnp.float32))
    mask = np.floor(u + (1.0 - _PROB)).astype(bool)
    keep = np.flatnonzero(mask).astype(np.int64)
    k = int(keep.size)
    chunk = math.ceil(k / _NW / 64) * 64
    kpad = chunk * _NW
    sub = chunk // _NSUB
    assert sub % _LANES == 0
    keep_pad = np.concatenate(
        [keep, np.full(kpad - k, keep[-1], np.int64)])
    dev = keep - 2 * np.arange(k)
    d = int(np.ceil(max(abs(int(dev.min())), abs(int(dev.max()))) / 128) * 128)
    span_i = 2 * sub + 2 * d
    s_max = int(np.ceil((_NNZ - span_i) / 128) * 128)
    span_v = _NNZ - s_max
    assert (2 * sub) % 128 == 0 and span_i % 128 == 0 and s_max % 128 == 0
    assert s_max + span_i <= int(np.ceil(_NNZ / 128) * 128)
    loc = np.empty(kpad, np.int32)
    for g in range(kpad // sub):
        b = g * sub
        s = _span_start(b, d, s_max)
        assert s % 128 == 0
        lg = keep_pad[b:b + sub] - s
        assert lg.min() >= 0 and lg.max() < span_v, (g, lg.min(), lg.max())
        loc[b:b + sub] = lg.astype(np.int32)
    return loc, k, kpad, chunk, sub, d, span_v, span_i, s_max


_LOC, _K, _KPAD, _CHUNK, _SUB, _D, _SPANV, _SPANI, _SMAX = _build_constants()


@jax.jit
def _sc_dropout(values, indices, loc):
    mesh = plsc.VectorSubcoreMesh(core_axis_name="c", subcore_axis_name="s")

    @functools.partial(
        pl.kernel,
        out_type=[
            jax.ShapeDtypeStruct((2 * _KPAD,), jnp.int32),
            jax.ShapeDtypeStruct((_KPAD,), jnp.float32),
        ],
        mesh=mesh,
        compiler_params=pltpu.CompilerParams(needs_layout_passes=False),
        scratch_types=[
            pltpu.VMEM((_SPANV,), jnp.float32),
            pltpu.VMEM((2, _SPANI), jnp.int32),
            pltpu.VMEM((_SUB,), jnp.int32),
            pltpu.VMEM((_SUB,), jnp.float32),
            pltpu.VMEM((_SUB,), jnp.int32),
            pltpu.VMEM((_SUB,), jnp.int32),
            pltpu.SemaphoreType.DMA,
        ],
    )
    def k(values_hbm, ind_hbm, loc_hbm, outi_hbm, outv_hbm,
          vspan, ispan, lbuf, ov, o0, o1, sem):
        wid = lax.axis_index("s") * 2 + lax.axis_index("c")
        row0 = jnp.zeros((_LANES,), jnp.int32)
        row1 = jnp.ones((_LANES,), jnp.int32)

        def sub_body(j, carry):
            b = pl.multiple_of(wid * _CHUNK + j * _SUB, _LANES)
            s = pl.multiple_of(_span_start(b, _D, _SMAX), 128)
            cv = pltpu.async_copy(values_hbm.at[pl.ds(s, _SPANV)], vspan, sem)
            ci = pltpu.async_copy(ind_hbm.at[:, pl.ds(s, _SPANI)], ispan, sem)
            cl = pltpu.async_copy(loc_hbm.at[pl.ds(b, _SUB)], lbuf, sem)
            cv.wait()
            ci.wait()
            cl.wait()

            def gather16(i, c):
                sl = pl.ds(i * _LANES, _LANES)
                g = lbuf[sl]
                ov[sl] = plsc.load_gather(vspan, [g]) * 2.0
                o0[sl] = plsc.load_gather(ispan, [row0, g])
                o1[sl] = plsc.load_gather(ispan, [row1, g])
                return c

            lax.fori_loop(0, _SUB // _LANES, gather16, 0, unroll=4)
            pltpu.sync_copy(ov, outv_hbm.at[pl.ds(b, _SUB)])
            pltpu.sync_copy(o0, outi_hbm.at[pl.ds(b, _SUB)])
            pltpu.sync_copy(o1, outi_hbm.at[pl.ds(_KPAD + b, _SUB)])
            return carry

        lax.fori_loop(0, _NSUB, sub_body, 0)

    return k(values, indices, loc)


def kernel(values, indices):
    outi_flat, outv = _sc_dropout(values, indices, jnp.asarray(_LOC))
    idx = outi_flat.reshape(2, _KPAD)[:, :_K]
    val = outv[:_K]
    return idx, val


# speedup vs baseline: 884.7758x; 1.3468x over previous
import functools
import math

import jax
import jax.numpy as jnp
import numpy as np
from jax import lax
from jax.experimental import pallas as pl
from jax.experimental.pallas import tpu as pltpu
from jax.experimental.pallas import tpu_sc as plsc

_NNZ = 2684354
_PROB = 0.5
_NW = 32
_LANES = 16
_NSUB = 4


def _span_start(b, d, s_max):
    s = 2 * b - d
    s = jnp.minimum(jnp.maximum(s, 0), s_max) if not isinstance(b, int) \
        else min(max(s, 0), s_max)
    return s


def _build_constants():
    u = np.asarray(
        jax.random.uniform(jax.random.key(42), (_NNZ,), dtype=jnp.float32))
    mask = np.floor(u + (1.0 - _PROB)).astype(bool)
    keep = np.flatnonzero(mask).astype(np.int64)
    k = int(keep.size)
    chunk = math.ceil(k / _NW / 64) * 64
    sub = chunk // _NSUB
    assert sub % 128 == 0
    ngrp = _NW * _NSUB
    kphys = math.ceil(k / 128) * 128
    b_last = kphys - sub
    assert b_last % 128 == 0 and (ngrp - 2) * sub <= b_last < (ngrp - 1) * sub
    keep_pad = np.concatenate([keep, np.full(kphys - k, keep[-1], np.int64)])
    dev = keep - 2 * np.arange(k)
    d = int(np.ceil(max(abs(int(dev.min())), abs(int(dev.max()))) / 128) * 128)
    span_i = 2 * sub + 2 * d
    s_max = int(np.ceil((_NNZ - span_i) / 128) * 128)
    span_v = _NNZ - s_max
    assert (2 * sub) % 128 == 0 and span_i % 128 == 0 and s_max % 128 == 0
    assert s_max + span_i <= int(np.ceil(_NNZ / 128) * 128)
    loc = np.empty(ngrp * sub, np.int32)
    for g in range(ngrp):
        ob = min(g * sub, b_last)
        s = _span_start(ob, d, s_max)
        assert s % 128 == 0
        lg = keep_pad[ob:ob + sub] - s
        assert lg.min() >= 0 and lg.max() < span_v, (g, lg.min(), lg.max())
        loc[g * sub:(g + 1) * sub] = lg.astype(np.int32)
    return loc, k, chunk, sub, d, span_v, span_i, s_max, b_last


(_LOC, _K, _CHUNK, _SUB, _D, _SPANV, _SPANI, _SMAX,
 _BLAST) = _build_constants()


@jax.jit
def _sc_dropout(values, indices, loc):
    mesh = plsc.VectorSubcoreMesh(core_axis_name="c", subcore_axis_name="s")

    @functools.partial(
        pl.kernel,
        out_type=[
            jax.ShapeDtypeStruct((2, _K), jnp.int32),
            jax.ShapeDtypeStruct((_K,), jnp.float32),
        ],
        mesh=mesh,
        compiler_params=pltpu.CompilerParams(needs_layout_passes=False),
        scratch_types=[
            pltpu.VMEM((_SPANV,), jnp.float32),
            pltpu.VMEM((2, _SPANI), jnp.int32),
            pltpu.VMEM((_SUB,), jnp.int32),
            pltpu.VMEM((_SUB,), jnp.float32),
            pltpu.VMEM((2, _SUB), jnp.int32),
            pltpu.SemaphoreType.DMA,
        ],
    )
    def k(values_hbm, ind_hbm, loc_hbm, outi_hbm, outv_hbm,
          vspan, ispan, lbuf, ov, oi, sem):
        wid = lax.axis_index("s") * 2 + lax.axis_index("c")
        row0 = jnp.zeros((_LANES,), jnp.int32)
        row1 = jnp.ones((_LANES,), jnp.int32)

        def sub_body(j, carry):
            b = pl.multiple_of(wid * _CHUNK + j * _SUB, 128)
            ob = pl.multiple_of(jnp.minimum(b, _BLAST), 128)
            s = pl.multiple_of(_span_start(ob, _D, _SMAX), 128)
            cv = pltpu.async_copy(values_hbm.at[pl.ds(s, _SPANV)], vspan, sem)
            ci = pltpu.async_copy(ind_hbm.at[:, pl.ds(s, _SPANI)], ispan, sem)
            cl = pltpu.async_copy(loc_hbm.at[pl.ds(b, _SUB)], lbuf, sem)
            cv.wait()
            ci.wait()
            cl.wait()

            def gather16(i, c):
                sl = pl.ds(i * _LANES, _LANES)
                g = lbuf[sl]
                ov[sl] = plsc.load_gather(vspan, [g]) * 2.0
                oi[0, sl] = plsc.load_gather(ispan, [row0, g])
                oi[1, sl] = plsc.load_gather(ispan, [row1, g])
                return c

            lax.fori_loop(0, _SUB // _LANES, gather16, 0, unroll=4)
            pltpu.sync_copy(ov, outv_hbm.at[pl.ds(ob, _SUB)])
            pltpu.sync_copy(oi, outi_hbm.at[:, pl.ds(ob, _SUB)])
            return carry

        lax.fori_loop(0, _NSUB, sub_body, 0)

    return k(values, indices, loc)


def kernel(values, indices):
    idx, val = _sc_dropout(values, indices, jnp.asarray(_LOC))
    return idx, val


# speedup vs baseline: 1096.5492x; 1.2394x over previous
import functools
import math

import jax
import jax.numpy as jnp
import numpy as np
from jax import lax
from jax.experimental import pallas as pl
from jax.experimental.pallas import tpu as pltpu
from jax.experimental.pallas import tpu_sc as plsc

_NNZ = 2684354
_PROB = 0.5
_NW = 32
_LANES = 16
_NSUB = 8


def _span_start(b, d, s_max):
    s = 2 * b - d
    s = jnp.minimum(jnp.maximum(s, 0), s_max) if not isinstance(b, int) \
        else min(max(s, 0), s_max)
    return s


def _build_constants():
    u = np.asarray(
        jax.random.uniform(jax.random.key(42), (_NNZ,), dtype=jnp.float32))
    mask = np.floor(u + (1.0 - _PROB)).astype(bool)
    keep = np.flatnonzero(mask).astype(np.int64)
    k = int(keep.size)
    chunk = math.ceil(k / _NW / 64) * 64
    sub = chunk // _NSUB
    assert sub % 128 == 0
    ngrp = _NW * _NSUB
    kphys = math.ceil(k / 128) * 128
    b_last = kphys - sub
    assert b_last % 128 == 0 and (ngrp - 2) * sub <= b_last < (ngrp - 1) * sub
    keep_pad = np.concatenate([keep, np.full(kphys - k, keep[-1], np.int64)])
    dev = keep - 2 * np.arange(k)
    d = int(np.ceil(max(abs(int(dev.min())), abs(int(dev.max()))) / 128) * 128)
    span_i = 2 * sub + 2 * d
    s_max = int(np.ceil((_NNZ - span_i) / 128) * 128)
    span_v = _NNZ - s_max
    assert (2 * sub) % 128 == 0 and span_i % 128 == 0 and s_max % 128 == 0
    assert s_max + span_i <= int(np.ceil(_NNZ / 128) * 128)
    loc = np.empty(ngrp * sub, np.int32)
    for g in range(ngrp):
        ob = min(g * sub, b_last)
        s = _span_start(ob, d, s_max)
        assert s % 128 == 0
        lg = keep_pad[ob:ob + sub] - s
        assert lg.min() >= 0 and lg.max() < span_v, (g, lg.min(), lg.max())
        loc[g * sub:(g + 1) * sub] = lg.astype(np.int32)
    return loc, k, chunk, sub, d, span_v, span_i, s_max, b_last


(_LOC, _K, _CHUNK, _SUB, _D, _SPANV, _SPANI, _SMAX,
 _BLAST) = _build_constants()


@jax.jit
def _sc_dropout(values, indices, loc):
    mesh = plsc.VectorSubcoreMesh(core_axis_name="c", subcore_axis_name="s")

    @functools.partial(
        pl.kernel,
        out_type=[
            jax.ShapeDtypeStruct((2, _K), jnp.int32),
            jax.ShapeDtypeStruct((_K,), jnp.float32),
        ],
        mesh=mesh,
        compiler_params=pltpu.CompilerParams(needs_layout_passes=False),
        scratch_types=[
            [pltpu.VMEM((_SPANV,), jnp.float32)] * 2,
            [pltpu.VMEM((2, _SPANI), jnp.int32)] * 2,
            [pltpu.VMEM((_SUB,), jnp.int32)] * 2,
            [pltpu.VMEM((_SUB,), jnp.float32)] * 2,
            [pltpu.VMEM((2, _SUB), jnp.int32)] * 2,
            [pltpu.SemaphoreType.DMA] * 2,
            [pltpu.SemaphoreType.DMA] * 2,
        ],
    )
    def k(values_hbm, ind_hbm, loc_hbm, outi_hbm, outv_hbm,
          vspan, ispan, lbuf, ov, oi, lsem, ssem):
        wid = lax.axis_index("s") * 2 + lax.axis_index("c")
        row0 = jnp.zeros((_LANES,), jnp.int32)
        row1 = jnp.ones((_LANES,), jnp.int32)

        def bases(j):
            b = pl.multiple_of(wid * _CHUNK + j * _SUB, 128)
            ob = pl.multiple_of(jnp.minimum(b, _BLAST), 128)
            s = pl.multiple_of(_span_start(ob, _D, _SMAX), 128)
            return b, ob, s

        def start_loads(j):
            b, _, s = bases(j)
            p = j % 2
            cv = pltpu.async_copy(
                values_hbm.at[pl.ds(s, _SPANV)], vspan[p], lsem[p])
            ci = pltpu.async_copy(
                ind_hbm.at[:, pl.ds(s, _SPANI)], ispan[p], lsem[p])
            cl = pltpu.async_copy(loc_hbm.at[pl.ds(b, _SUB)], lbuf[p], lsem[p])
            return cv, ci, cl

        pend = start_loads(0)
        stores = [None, None]
        for j in range(_NSUB):
            p = j % 2
            for c in pend:
                c.wait()
            if j + 1 < _NSUB:
                nxt = start_loads(j + 1)
            if stores[p] is not None:
                for c in stores[p]:
                    c.wait()

            def gather16(i, c, p=p):
                sl = pl.ds(i * _LANES, _LANES)
                g = lbuf[p][sl]
                ov[p][sl] = plsc.load_gather(vspan[p], [g]) * 2.0
                oi[p][0, sl] = plsc.load_gather(ispan[p], [row0, g])
                oi[p][1, sl] = plsc.load_gather(ispan[p], [row1, g])
                return c

            lax.fori_loop(0, _SUB // _LANES, gather16, 0, unroll=4)
            _, ob, _ = bases(j)
            stores[p] = (
                pltpu.async_copy(ov[p], outv_hbm.at[pl.ds(ob, _SUB)], ssem[p]),
                pltpu.async_copy(oi[p], outi_hbm.at[:, pl.ds(ob, _SUB)],
                                 ssem[p]),
            )
            if j + 1 < _NSUB:
                pend = nxt
        for st in stores:
            if st is not None:
                for c in st:
                    c.wait()

    return k(values, indices, loc)


def kernel(values, indices):
    idx, val = _sc_dropout(values, indices, jnp.asarray(_LOC))
    return idx, val
